# Initial kernel scaffold; baseline (speedup 1.0000x reference)
import jax
import jax.numpy as jnp
from jax.experimental import pallas as pl


def kernel(original_kpts, segment):
    raise NotImplementedError("write your pallas kernel here")


# speedup vs baseline: 1.1684x; 1.1684x over previous
import dataclasses
import functools

import jax
import jax.numpy as jnp
from jax import lax
from jax.experimental import pallas as pl
from jax.experimental.pallas import tpu as pltpu
from jax.experimental.pallas import tpu_sc as plsc

B, M, C, H, W = 8, 2048, 192, 224, 224
HW = H * W

NC, NS, L = 2, 16, 16
NW = NC * NS
KPW = (B * M) // NW
CK = 128
SLICE = 128
NSL = (CK * C) // SLICE
FIRE = 8

_mesh = plsc.VectorSubcoreMesh(core_axis_name="c", subcore_axis_name="s")

_cp = pltpu.CompilerParams()
if "needs_layout_passes" in pltpu.CompilerParams.__dataclass_fields__:
    _cp = dataclasses.replace(_cp, needs_layout_passes=False)


@functools.partial(
    pl.kernel,
    mesh=_mesh,
    compiler_params=_cp,
    out_type=jax.ShapeDtypeStruct((B * M * C,), jnp.float32),
    scratch_types=[
        pltpu.VMEM((CK * 2,), jnp.float32),
        pltpu.VMEM((CK,), jnp.int32),
        pltpu.VMEM((CK * C,), jnp.int32),
        pltpu.VMEM((CK * C,), jnp.float32),
        pltpu.SemaphoreType.DMA,
    ],
)
def _gather_si(kpts_hbm, seg_hbm, out_hbm, kv, bv, iv, gv, sem):
    wid = lax.axis_index("s") * NC + lax.axis_index("c")
    kp0 = wid * KPW
    b = kp0 // M
    bbase = b * (C * HW)

    lanes = lax.iota(jnp.int32, L)

    @pl.loop(0, KPW // CK)
    def _chunk(t):
        row0 = kp0 + t * CK
        pltpu.sync_copy(kpts_hbm.at[pl.ds(row0 * 2, CK * 2)], kv)

        for j in range(CK // L):
            ev = lanes * 2 + (j * 2 * L)
            xf = plsc.load_gather(kv, [ev]) * float(H)
            yf = plsc.load_gather(kv, [ev + 1]) * float(W)
            xi = xf.astype(jnp.int32)
            xi = jnp.where(xi.astype(jnp.float32) > xf, xi - 1, xi)
            yi = yf.astype(jnp.int32)
            yi = jnp.where(yi.astype(jnp.float32) > yf, yi - 1, yi)
            p = jnp.minimum(jnp.maximum(xi * W + yi, 0), HW - 1)
            bv[pl.ds(j * L, L)] = p + bbase

        for j in range(CK // L):
            base16 = bv[pl.ds(j * L, L)]
            pos0 = lanes * C + (j * L * C)

            @pl.loop(0, C)
            def _expand(ch):
                plsc.store_scatter(iv, [pos0 + ch], base16 + ch * HW)

        for g in range(0, NSL, FIRE):
            cps = [
                pltpu.async_copy(
                    seg_hbm.at[iv.at[pl.ds((g + r) * SLICE, SLICE)]],
                    gv.at[pl.ds((g + r) * SLICE, SLICE)],
                    sem,
                )
                for r in range(FIRE)
            ]
            for cp in cps:
                cp.wait()

        pltpu.sync_copy(gv, out_hbm.at[pl.ds(row0 * C, CK * C)])


def kernel(original_kpts, segment):
    out = _gather_si(original_kpts.reshape(-1), segment.reshape(-1))
    return out.reshape(B, M, C)


# speedup vs baseline: 1.2787x; 1.0944x over previous
import dataclasses
import functools

import jax
import jax.numpy as jnp
from jax import lax
from jax.experimental import pallas as pl
from jax.experimental.pallas import tpu as pltpu
from jax.experimental.pallas import tpu_sc as plsc

B, M, C, H, W = 8, 2048, 192, 224, 224
HW = H * W

NC, NS, L = 2, 16, 16
NW = NC * NS
KPW = (B * M) // NW
CK = 128
SLICE = 2048
NSL = (CK * C) // SLICE
FIRE = 4

_mesh = plsc.VectorSubcoreMesh(core_axis_name="c", subcore_axis_name="s")

_cp = pltpu.CompilerParams()
if "needs_layout_passes" in pltpu.CompilerParams.__dataclass_fields__:
    _cp = dataclasses.replace(_cp, needs_layout_passes=False)


@functools.partial(
    pl.kernel,
    mesh=_mesh,
    compiler_params=_cp,
    out_type=jax.ShapeDtypeStruct((B * M * C,), jnp.float32),
    scratch_types=[
        pltpu.VMEM((CK * 2,), jnp.float32),
        pltpu.VMEM((CK,), jnp.int32),
        pltpu.VMEM((CK * C,), jnp.int32),
        pltpu.VMEM((CK * C,), jnp.float32),
        pltpu.SemaphoreType.DMA,
    ],
)
def _gather_si(kpts_hbm, seg_hbm, out_hbm, kv, bv, iv, gv, sem):
    wid = lax.axis_index("s") * NC + lax.axis_index("c")
    kp0 = wid * KPW
    b = kp0 // M
    bbase = b * (C * HW)

    lanes = lax.iota(jnp.int32, L)

    @pl.loop(0, KPW // CK)
    def _chunk(t):
        row0 = kp0 + t * CK
        pltpu.sync_copy(kpts_hbm.at[pl.ds(row0 * 2, CK * 2)], kv)

        for j in range(CK // L):
            ev = lanes * 2 + (j * 2 * L)
            xf = plsc.load_gather(kv, [ev]) * float(H)
            yf = plsc.load_gather(kv, [ev + 1]) * float(W)
            xi = xf.astype(jnp.int32)
            xi = jnp.where(xi.astype(jnp.float32) > xf, xi - 1, xi)
            yi = yf.astype(jnp.int32)
            yi = jnp.where(yi.astype(jnp.float32) > yf, yi - 1, yi)
            p = jnp.minimum(jnp.maximum(xi * W + yi, 0), HW - 1)
            bv[pl.ds(j * L, L)] = p + bbase

        for j in range(CK // L):
            base16 = bv[pl.ds(j * L, L)]
            pos0 = lanes * C + (j * L * C)

            @pl.loop(0, C)
            def _expand(ch):
                plsc.store_scatter(iv, [pos0 + ch], base16 + ch * HW)

        for g in range(0, NSL, FIRE):
            cps = [
                pltpu.async_copy(
                    seg_hbm.at[iv.at[pl.ds((g + r) * SLICE, SLICE)]],
                    gv.at[pl.ds((g + r) * SLICE, SLICE)],
                    sem,
                )
                for r in range(FIRE)
            ]
            for cp in cps:
                cp.wait()

        pltpu.sync_copy(gv, out_hbm.at[pl.ds(row0 * C, CK * C)])


def kernel(original_kpts, segment):
    out = _gather_si(original_kpts.reshape(-1), segment.reshape(-1))
    return out.reshape(B, M, C)
